# baseline (speedup 1.0000x reference)
import functools

import jax
import jax.numpy as jnp
from jax import lax
from jax.experimental import pallas as pl
from jax.experimental.pallas import tpu as pltpu
from jax.experimental.pallas import tpu_sc as plsc

NU = 50000
NI = 50000
N = NU + NI
D = 64
NP = 100352
HALF = NP // 2
S = HALF // 2
DUMMY = S
RPT = HALF // 16
CRPT = S // 16

E = 3200000
EPAD = 98 * 32768
EROWS = EPAD // 128
NBLK = 98
NBLK_L = 196
EPT = EPAD // 16
SENT = 1 << 28

B = 16384
F32 = jnp.float32

_mesh = plsc.VectorSubcoreMesh(core_axis_name="c", subcore_axis_name="s")


def _deg_body(cols_ref, deg_ref, deg_sh, cbuf, idxb, ones, dbuf):
    c = lax.axis_index("c")
    t = lax.axis_index("s")
    base = c * HALF

    for k in range(8):
        ones[pl.ds(k * 16, 16)] = jnp.full((16,), 1.0, F32)
    for k in range(28):
        dbuf[pl.ds(k * 16, 16)] = jnp.zeros((16,), F32)
    for blk in range(7):
        pltpu.sync_copy(dbuf, deg_sh.at[pl.ds(t * RPT + blk * 448, 448)])
    plsc.subcore_barrier()

    def blk_fn(b, carry):
        pltpu.sync_copy(cols_ref.at[pl.ds(t * (NBLK * 16) + b * 16, 16)], cbuf)
        for j in range(16):
            for k in range(8):
                cv = cbuf[j, pl.ds(k * 16, 16)]
                m = (cv >= base) & (cv < base + HALF)
                idxb[j, pl.ds(k * 16, 16)] = jnp.where(m, cv - base, HALF)
        for j in range(16):
            pltpu.sync_copy(ones, deg_sh.at[idxb.at[j]], add=True)
        return carry

    lax.fori_loop(0, NBLK, blk_fn, 0)
    plsc.subcore_barrier()

    for blk in range(7):
        off = t * RPT + blk * 448
        pltpu.sync_copy(deg_sh.at[pl.ds(off, 448)], dbuf)
        pltpu.sync_copy(dbuf, deg_ref.at[pl.ds(base + off, 448)])


_deg = pl.kernel(
    _deg_body,
    compiler_params=pltpu.CompilerParams(use_tc_tiling_on_sc=False),
    out_type=jax.ShapeDtypeStruct((NP,), F32),
    mesh=_mesh,
    scratch_types=[
        pltpu.VMEM_SHARED((HALF + 16,), F32),
        pltpu.VMEM((16, 128), jnp.int32),
        pltpu.VMEM((16, 128), jnp.int32),
        pltpu.VMEM((128,), F32),
        pltpu.VMEM((448,), F32),
    ],
)


def _scale_body(deg_ref, x0_ref, dinv_ref, y0_ref):
    d = deg_ref[...]
    di = jnp.where(d > 0.0, lax.rsqrt(d), 0.0)
    dinv_ref[...] = di
    y0_ref[...] = x0_ref[...] * di


def _scale(deg, x0):
    dblk = pl.BlockSpec((512, 1), lambda i: (i, 0))
    xblk = pl.BlockSpec((512, D), lambda i: (i, 0))
    return pl.pallas_call(
        _scale_body,
        grid=(NP // 512,),
        in_specs=[dblk, xblk],
        out_specs=[dblk, xblk],
        out_shape=[jax.ShapeDtypeStruct((NP, 1), F32),
                   jax.ShapeDtypeStruct((NP, D), F32)],
    )(deg, x0)


def _layer_body(y_ref, rows_ref, cols_ref, dinv_ref, y_out, x_out,
                accum, cbuf, rbufA, rbufB, idxA, idxB,
                g0, g1, abuf, xbuf, ybuf, dvb,
                semg0, semg1, sems0, sems1):
    c = lax.axis_index("c")
    t = lax.axis_index("s")

    def zwait(sem):
        pltpu.make_async_copy(y_ref.at[rbufB.at[7]], g1, sem).wait()

    for chunk in range(2):
        base = c * HALF + chunk * S

        for r in range(32):
            for k in range(4):
                ybuf[r, pl.ds(k * 16, 16)] = jnp.zeros((16,), F32)

        def zero_fn(blk, carry):
            pltpu.sync_copy(ybuf, accum.at[pl.ds(t * CRPT + blk * 32, 32)])
            return carry

        lax.fori_loop(0, 49, zero_fn, 0)

        for k in range(8):
            rbufB[7, pl.ds(k * 16, 16)] = jnp.zeros((16,), jnp.int32)
            idxB[7, pl.ds(k * 16, 16)] = jnp.full((16,), DUMMY, jnp.int32)
        plsc.subcore_barrier()
        pltpu.async_copy(y_ref.at[rbufB.at[7]], g1, semg1)
        pltpu.async_copy(g0, accum.at[idxB.at[7]], sems0, add=True)

        gpair = ((g0, semg0, sems0), (g1, semg1, sems1))

        def grp_fn(gb, carry):
            fc = 0
            prev = (g1, idxB, 7)
            for blk in range(4):
                rX, iX = (rbufA, idxA) if blk % 2 == 0 else (rbufB, idxB)
                off8 = t * (EPT // 128) + (gb * 4 + blk) * 8
                pltpu.sync_copy(cols_ref.at[pl.ds(off8, 8)], cbuf)
                pltpu.sync_copy(rows_ref.at[pl.ds(off8, 8)], rX)
                for j in range(8):
                    for k in range(8):
                        cv = cbuf[j, pl.ds(k * 16, 16)]
                        m = (cv >= base) & (cv < base + S)
                        iX[j, pl.ds(k * 16, 16)] = jnp.where(
                            m, cv - base, DUMMY)
                for j in range(8):
                    p = fc & 1
                    g, sg, ss = gpair[p]
                    pg, psg, pss = gpair[1 - p]
                    prev_g, prev_buf, prev_j = prev
                    zwait(ss)
                    pltpu.async_copy(y_ref.at[rX.at[j]], g, sg)
                    zwait(psg)
                    pltpu.async_copy(prev_g, accum.at[prev_buf.at[prev_j]],
                                     pss, add=True)
                    prev = (g, iX, j)
                    fc += 1
            return carry

        lax.fori_loop(0, NBLK_L // 4, grp_fn, 0)

        zwait(semg1)
        pltpu.async_copy(g1, accum.at[idxB.at[7]], sems1, add=True)
        zwait(sems0)
        zwait(sems1)
        plsc.subcore_barrier()

        def flush_fn(blk, carry):
            roff = t * CRPT + blk * 32
            noff = base + roff
            pltpu.sync_copy(accum.at[pl.ds(roff, 32)], abuf)
            pltpu.sync_copy(dinv_ref.at[pl.ds(noff, 32)], dvb)
            for vg in range(2):
                dvec = dvb[pl.ds(vg * 16, 16)]
                for lane in range(16):
                    dv = jnp.full((16,), dvec[lane], F32)
                    r = vg * 16 + lane
                    for k in range(4):
                        a = abuf[r, pl.ds(k * 16, 16)]
                        x = a * dv
                        xbuf[r, pl.ds(k * 16, 16)] = x
                        ybuf[r, pl.ds(k * 16, 16)] = x * dv
            pltpu.sync_copy(xbuf, x_out.at[pl.ds(noff, 32)])
            pltpu.sync_copy(ybuf, y_out.at[pl.ds(noff, 32)])
            return carry

        lax.fori_loop(0, 49, flush_fn, 0)
        plsc.subcore_barrier()


_layer = pl.kernel(
    _layer_body,
    compiler_params=pltpu.CompilerParams(use_tc_tiling_on_sc=False),
    out_type=[jax.ShapeDtypeStruct((NP, D), F32),
              jax.ShapeDtypeStruct((NP, D), F32)],
    mesh=_mesh,
    scratch_types=[
        pltpu.VMEM_SHARED((S + 16, D), F32),
        pltpu.VMEM((8, 128), jnp.int32),
        pltpu.VMEM((8, 128), jnp.int32),
        pltpu.VMEM((8, 128), jnp.int32),
        pltpu.VMEM((8, 128), jnp.int32),
        pltpu.VMEM((8, 128), jnp.int32),
        pltpu.VMEM((128, D), F32),
        pltpu.VMEM((128, D), F32),
        pltpu.VMEM((32, D), F32),
        pltpu.VMEM((32, D), F32),
        pltpu.VMEM((32, D), F32),
        pltpu.VMEM((32,), F32),
        pltpu.SemaphoreType.DMA,
        pltpu.SemaphoreType.DMA,
        pltpu.SemaphoreType.DMA,
        pltpu.SemaphoreType.DMA,
    ],
)


def _gather_body(final_ref, uids_ref, iids_ref, urows_ref, irows_ref,
                 idb, idxb, gbuf, sem):
    w = lax.axis_index("s") * 2 + lax.axis_index("c")

    pltpu.sync_copy(uids_ref.at[pl.ds(w * 4, 4)], idb)
    for j in range(4):
        pltpu.async_copy(final_ref.at[idb.at[j]], gbuf, sem).wait()
        pltpu.sync_copy(gbuf, urows_ref.at[pl.ds(w * 512 + j * 128, 128)])

    pltpu.sync_copy(iids_ref.at[pl.ds(w * 4, 4)], idb)
    for j in range(4):
        for k in range(8):
            idxb[j, pl.ds(k * 16, 16)] = idb[j, pl.ds(k * 16, 16)] + NU
    for j in range(4):
        pltpu.async_copy(final_ref.at[idxb.at[j]], gbuf, sem).wait()
        pltpu.sync_copy(gbuf, irows_ref.at[pl.ds(w * 512 + j * 128, 128)])


_gather = pl.kernel(
    _gather_body,
    compiler_params=pltpu.CompilerParams(use_tc_tiling_on_sc=False),
    out_type=[jax.ShapeDtypeStruct((B, D), F32),
              jax.ShapeDtypeStruct((B, D), F32)],
    mesh=_mesh,
    scratch_types=[
        pltpu.VMEM((4, 128), jnp.int32),
        pltpu.VMEM((4, 128), jnp.int32),
        pltpu.VMEM((128, D), F32),
        pltpu.SemaphoreType.DMA,
    ],
)


def _sum_body(a_ref, b_ref, c_ref, d_ref, o_ref):
    o_ref[...] = (a_ref[...] + b_ref[...] + c_ref[...] + d_ref[...]) * 0.25


def _mean4(x0, x1, x2, x3):
    blk = pl.BlockSpec((512, D), lambda i: (i, 0))
    return pl.pallas_call(
        _sum_body,
        grid=(NP // 512,),
        in_specs=[blk, blk, blk, blk],
        out_specs=blk,
        out_shape=jax.ShapeDtypeStruct((NP, D), F32),
    )(x0, x1, x2, x3)


def _dot_body(u_ref, i_ref, o_ref):
    s = jnp.sum(u_ref[...] * i_ref[...], axis=1, keepdims=True)
    o_ref[...] = jax.nn.sigmoid(s)


def _dot(u, i):
    blk = pl.BlockSpec((2048, D), lambda b: (b, 0))
    oblk = pl.BlockSpec((2048, 1), lambda b: (b, 0))
    return pl.pallas_call(
        _dot_body,
        grid=(B // 2048,),
        in_specs=[blk, blk],
        out_specs=oblk,
        out_shape=jax.ShapeDtypeStruct((B, 1), F32),
    )(u, i)


def kernel(user_ids, item_ids, edge_index, user_embedding, item_embedding):
    user_ids = user_ids.astype(jnp.int32)
    item_ids = item_ids.astype(jnp.int32)
    edge_index = edge_index.astype(jnp.int32)

    x0 = jnp.concatenate([user_embedding, item_embedding], axis=0)
    x0 = jnp.pad(x0, ((0, NP - N), (0, 0)))
    rows2 = jnp.pad(edge_index[0], (0, EPAD - E)).reshape(EROWS, 128)
    cols2 = jnp.pad(edge_index[1], (0, EPAD - E),
                    constant_values=SENT).reshape(EROWS, 128)
    uids2 = user_ids.reshape(B // 128, 128)
    iids2 = item_ids.reshape(B // 128, 128)

    deg = _deg(cols2)
    dinv2, y = _scale(deg.reshape(NP, 1), x0)
    dinv = dinv2.reshape(NP)
    y, x1 = _layer(y, rows2, cols2, dinv)
    y, x2 = _layer(y, rows2, cols2, dinv)
    _, x3 = _layer(y, rows2, cols2, dinv)

    final = _mean4(x0, x1, x2, x3)
    ur, ir = _gather(final, uids2, iids2)
    return _dot(ur, ir)


# speedup vs baseline: 1.9209x; 1.9209x over previous
import functools

import jax
import jax.numpy as jnp
from jax import lax
from jax.experimental import pallas as pl
from jax.experimental.pallas import tpu as pltpu
from jax.experimental.pallas import tpu_sc as plsc

NU = 50000
NI = 50000
N = NU + NI
D = 64
NP = 100352
HALF = NP // 2
S = HALF // 2
DUMMY = S
RPT = HALF // 16
CRPT = S // 16

E = 3200000
EPAD = 98 * 32768
EROWS = EPAD // 128
NBLK = 98
NBLK_L = 196
EPT = EPAD // 16
SENT = 1 << 28

B = 16384
F32 = jnp.float32

_mesh = plsc.VectorSubcoreMesh(core_axis_name="c", subcore_axis_name="s")


def _deg_body(cols_ref, deg_ref, deg_sh, cbuf, idxb, ones, dbuf):
    c = lax.axis_index("c")
    t = lax.axis_index("s")
    base = c * HALF

    for k in range(8):
        ones[pl.ds(k * 16, 16)] = jnp.full((16,), 1.0, F32)
    for k in range(28):
        dbuf[pl.ds(k * 16, 16)] = jnp.zeros((16,), F32)
    for blk in range(7):
        pltpu.sync_copy(dbuf, deg_sh.at[pl.ds(t * RPT + blk * 448, 448)])
    plsc.subcore_barrier()

    def blk_fn(b, carry):
        pltpu.sync_copy(cols_ref.at[pl.ds(t * (NBLK * 16) + b * 16, 16)], cbuf)
        for j in range(16):
            for k in range(8):
                cv = cbuf[j, pl.ds(k * 16, 16)]
                m = (cv >= base) & (cv < base + HALF)
                idxb[j, pl.ds(k * 16, 16)] = jnp.where(m, cv - base, HALF)
        for j in range(16):
            pltpu.sync_copy(ones, deg_sh.at[idxb.at[j]], add=True)
        return carry

    lax.fori_loop(0, NBLK, blk_fn, 0)
    plsc.subcore_barrier()

    for blk in range(7):
        off = t * RPT + blk * 448
        pltpu.sync_copy(deg_sh.at[pl.ds(off, 448)], dbuf)
        pltpu.sync_copy(dbuf, deg_ref.at[pl.ds(base + off, 448)])


_deg = pl.kernel(
    _deg_body,
    compiler_params=pltpu.CompilerParams(use_tc_tiling_on_sc=False),
    out_type=jax.ShapeDtypeStruct((NP,), F32),
    mesh=_mesh,
    scratch_types=[
        pltpu.VMEM_SHARED((HALF + 16,), F32),
        pltpu.VMEM((16, 128), jnp.int32),
        pltpu.VMEM((16, 128), jnp.int32),
        pltpu.VMEM((128,), F32),
        pltpu.VMEM((448,), F32),
    ],
)


def _scale_body(deg_ref, x0_ref, dinv_ref, y0_ref):
    d = deg_ref[...]
    di = jnp.where(d > 0.0, lax.rsqrt(d), 0.0)
    dinv_ref[...] = di
    y0_ref[...] = x0_ref[...] * di


def _scale(deg, x0):
    dblk = pl.BlockSpec((512, 1), lambda i: (i, 0))
    xblk = pl.BlockSpec((512, D), lambda i: (i, 0))
    return pl.pallas_call(
        _scale_body,
        grid=(NP // 512,),
        in_specs=[dblk, xblk],
        out_specs=[dblk, xblk],
        out_shape=[jax.ShapeDtypeStruct((NP, 1), F32),
                   jax.ShapeDtypeStruct((NP, D), F32)],
    )(deg, x0)


PCAP = 1424
TRASH = 1408
FB = 128


def _layer_body(y_ref, rows_ref, cols_ref, dinv_ref, y_out, x_out,
                accum, cbuf, rbuf, pcol, prow, tailb, sbuf,
                cf0, cf1, rf0, rf1, g0, g1, abuf, xbuf, ybuf, dvb,
                semg0, semg1, sems0, sems1):
    c = lax.axis_index("c")
    t = lax.axis_index("s")
    iota = jax.lax.iota(jnp.int32, 16)
    for k in range(2):
        sbuf[pl.ds(k * 16, 16)] = jnp.zeros((16,), jnp.int32)

    def prefix16(b):
        s = b
        for sh in (1, 2, 4, 8):
            sbuf[pl.ds(16, 16)] = s
            s = s + sbuf[pl.ds(16 - sh, 16)]
        return s

    def zwait(sem):
        pltpu.make_async_copy(y_ref.at[rf0], g0, sem).wait()

    bufs = ((rf0, cf0, g0, semg0, sems0), (rf1, cf1, g1, semg1, sems1))

    def fire(p, foff):
        rf, cf, g, sg, ss = bufs[p]
        rfq, cfq, gq, sgq, ssq = bufs[1 - p]
        zwait(ss)
        for k in range(8):
            rf[pl.ds(k * 16, 16)] = prow[pl.ds(foff * FB + k * 16, 16)]
            cf[pl.ds(k * 16, 16)] = pcol[pl.ds(foff * FB + k * 16, 16)]
        pltpu.async_copy(y_ref.at[rf], g, sg)
        zwait(sgq)
        pltpu.async_copy(gq, accum.at[cfq], ssq, add=True)

    for chunk in range(2):
        base = c * HALF + chunk * S

        for r in range(32):
            for k in range(4):
                ybuf[r, pl.ds(k * 16, 16)] = jnp.zeros((16,), F32)

        def zero_fn(blk, carry):
            pltpu.sync_copy(ybuf, accum.at[pl.ds(t * CRPT + blk * 32, 32)])
            return carry

        lax.fori_loop(0, 49, zero_fn, 0)

        for k in range(8):
            z16 = jnp.zeros((16,), jnp.int32)
            d16 = jnp.full((16,), DUMMY, jnp.int32)
            rf0[pl.ds(k * 16, 16)] = z16
            rf1[pl.ds(k * 16, 16)] = z16
            cf0[pl.ds(k * 16, 16)] = d16
            cf1[pl.ds(k * 16, 16)] = d16
        plsc.subcore_barrier()
        pltpu.async_copy(g0, accum.at[cf0], sems0, add=True)
        pltpu.async_copy(y_ref.at[rf1], g1, semg1)

        def blk_fn(b, carry):
            cnt, foff, gcnt = carry
            off8 = t * (EPT // 128) + b * 8
            pltpu.sync_copy(cols_ref.at[pl.ds(off8, 8)], cbuf)
            pltpu.sync_copy(rows_ref.at[pl.ds(off8, 8)], rbuf)
            for grp in range(16):
                for v in range(4):
                    j, k = divmod(grp * 4 + v, 8)
                    cv = cbuf[j, pl.ds(k * 16, 16)]
                    rv = rbuf[j, pl.ds(k * 16, 16)]
                    m = (cv >= base) & (cv < base + S)
                    csum = prefix16(jnp.where(m, 1, 0))
                    pos = jnp.where(m, cnt + csum - 1, TRASH)
                    plsc.store_scatter(pcol, [pos], cv - base)
                    plsc.store_scatter(prow, [pos], rv)
                    cnt = cnt + csum[15]
                do_fire = cnt - foff * FB >= FB
                even = gcnt % 2 == 0

                @pl.when(do_fire & even)
                def _():
                    fire(0, foff)

                @pl.when(do_fire & (~even))
                def _():
                    fire(1, foff)

                foff = jnp.where(do_fire, foff + 1, foff)
                gcnt = jnp.where(do_fire, gcnt + 1, gcnt)
            lo = foff * FB

            @pl.when(foff > 0)
            def _():
                for k in range(12):
                    rt = prow[pl.ds(lo + k * 16, 16)]
                    ct = pcol[pl.ds(lo + k * 16, 16)]
                    prow[pl.ds(k * 16, 16)] = rt
                    pcol[pl.ds(k * 16, 16)] = ct

            return (cnt - lo, jnp.int32(0), gcnt)

        cnt, _, gcnt = lax.fori_loop(
            0, NBLK_L, blk_fn,
            (jnp.int32(0), jnp.int32(0), jnp.int32(0)))

        for v in range(16):
            mr = (v * 16 + iota) < cnt
            rv = prow[pl.ds(v * 16, 16)]
            prow[pl.ds(v * 16, 16)] = jnp.where(mr, rv, 0)
            cv = pcol[pl.ds(v * 16, 16)]
            pcol[pl.ds(v * 16, 16)] = jnp.where(mr, cv, DUMMY)
        even = gcnt % 2 == 0

        @pl.when(even)
        def _():
            fire(0, 0)
            fire(1, 1)
            zwait(semg1)
            pltpu.async_copy(g1, accum.at[cf1], sems1, add=True)

        @pl.when(~even)
        def _():
            fire(1, 0)
            fire(0, 1)
            zwait(semg0)
            pltpu.async_copy(g0, accum.at[cf0], sems0, add=True)

        zwait(sems0)
        zwait(sems1)
        plsc.subcore_barrier()

        def flush_fn(blk, carry):
            roff = t * CRPT + blk * 32
            noff = base + roff
            pltpu.sync_copy(accum.at[pl.ds(roff, 32)], abuf)
            pltpu.sync_copy(dinv_ref.at[pl.ds(noff, 32)], dvb)
            for vg in range(2):
                dvec = dvb[pl.ds(vg * 16, 16)]
                for lane in range(16):
                    dv = jnp.full((16,), dvec[lane], F32)
                    r = vg * 16 + lane
                    for k in range(4):
                        a = abuf[r, pl.ds(k * 16, 16)]
                        x = a * dv
                        xbuf[r, pl.ds(k * 16, 16)] = x
                        ybuf[r, pl.ds(k * 16, 16)] = x * dv
            pltpu.sync_copy(xbuf, x_out.at[pl.ds(noff, 32)])
            pltpu.sync_copy(ybuf, y_out.at[pl.ds(noff, 32)])
            return carry

        lax.fori_loop(0, 49, flush_fn, 0)
        plsc.subcore_barrier()


_layer = pl.kernel(
    _layer_body,
    compiler_params=pltpu.CompilerParams(use_tc_tiling_on_sc=False,
                                         needs_layout_passes=False),
    out_type=[jax.ShapeDtypeStruct((NP, D), F32),
              jax.ShapeDtypeStruct((NP, D), F32)],
    mesh=_mesh,
    scratch_types=[
        pltpu.VMEM_SHARED((S + 16, D), F32),
        pltpu.VMEM((8, 128), jnp.int32),
        pltpu.VMEM((8, 128), jnp.int32),
        pltpu.VMEM((PCAP,), jnp.int32),
        pltpu.VMEM((PCAP,), jnp.int32),
        pltpu.VMEM((192,), jnp.int32),
        pltpu.VMEM((32,), jnp.int32),
        pltpu.VMEM((128,), jnp.int32),
        pltpu.VMEM((128,), jnp.int32),
        pltpu.VMEM((128,), jnp.int32),
        pltpu.VMEM((128,), jnp.int32),
        pltpu.VMEM((FB, D), F32),
        pltpu.VMEM((FB, D), F32),
        pltpu.VMEM((32, D), F32),
        pltpu.VMEM((32, D), F32),
        pltpu.VMEM((32, D), F32),
        pltpu.VMEM((32,), F32),
        pltpu.SemaphoreType.DMA,
        pltpu.SemaphoreType.DMA,
        pltpu.SemaphoreType.DMA,
        pltpu.SemaphoreType.DMA,
    ],
)


def _gather_body(final_ref, uids_ref, iids_ref, urows_ref, irows_ref,
                 idb, idxb, gbuf, sem):
    w = lax.axis_index("s") * 2 + lax.axis_index("c")

    pltpu.sync_copy(uids_ref.at[pl.ds(w * 4, 4)], idb)
    for j in range(4):
        pltpu.async_copy(final_ref.at[idb.at[j]], gbuf, sem).wait()
        pltpu.sync_copy(gbuf, urows_ref.at[pl.ds(w * 512 + j * 128, 128)])

    pltpu.sync_copy(iids_ref.at[pl.ds(w * 4, 4)], idb)
    for j in range(4):
        for k in range(8):
            idxb[j, pl.ds(k * 16, 16)] = idb[j, pl.ds(k * 16, 16)] + NU
    for j in range(4):
        pltpu.async_copy(final_ref.at[idxb.at[j]], gbuf, sem).wait()
        pltpu.sync_copy(gbuf, irows_ref.at[pl.ds(w * 512 + j * 128, 128)])


_gather = pl.kernel(
    _gather_body,
    compiler_params=pltpu.CompilerParams(use_tc_tiling_on_sc=False),
    out_type=[jax.ShapeDtypeStruct((B, D), F32),
              jax.ShapeDtypeStruct((B, D), F32)],
    mesh=_mesh,
    scratch_types=[
        pltpu.VMEM((4, 128), jnp.int32),
        pltpu.VMEM((4, 128), jnp.int32),
        pltpu.VMEM((128, D), F32),
        pltpu.SemaphoreType.DMA,
    ],
)


def _sum_body(a_ref, b_ref, c_ref, d_ref, o_ref):
    o_ref[...] = (a_ref[...] + b_ref[...] + c_ref[...] + d_ref[...]) * 0.25


def _mean4(x0, x1, x2, x3):
    blk = pl.BlockSpec((512, D), lambda i: (i, 0))
    return pl.pallas_call(
        _sum_body,
        grid=(NP // 512,),
        in_specs=[blk, blk, blk, blk],
        out_specs=blk,
        out_shape=jax.ShapeDtypeStruct((NP, D), F32),
    )(x0, x1, x2, x3)


def _dot_body(u_ref, i_ref, o_ref):
    s = jnp.sum(u_ref[...] * i_ref[...], axis=1, keepdims=True)
    o_ref[...] = jax.nn.sigmoid(s)


def _dot(u, i):
    blk = pl.BlockSpec((2048, D), lambda b: (b, 0))
    oblk = pl.BlockSpec((2048, 1), lambda b: (b, 0))
    return pl.pallas_call(
        _dot_body,
        grid=(B // 2048,),
        in_specs=[blk, blk],
        out_specs=oblk,
        out_shape=jax.ShapeDtypeStruct((B, 1), F32),
    )(u, i)


def kernel(user_ids, item_ids, edge_index, user_embedding, item_embedding):
    user_ids = user_ids.astype(jnp.int32)
    item_ids = item_ids.astype(jnp.int32)
    edge_index = edge_index.astype(jnp.int32)

    x0 = jnp.concatenate([user_embedding, item_embedding], axis=0)
    x0 = jnp.pad(x0, ((0, NP - N), (0, 0)))
    rows2 = jnp.pad(edge_index[0], (0, EPAD - E)).reshape(EROWS, 128)
    cols2 = jnp.pad(edge_index[1], (0, EPAD - E),
                    constant_values=SENT).reshape(EROWS, 128)
    uids2 = user_ids.reshape(B // 128, 128)
    iids2 = item_ids.reshape(B // 128, 128)

    deg = _deg(cols2)
    dinv2, y = _scale(deg.reshape(NP, 1), x0)
    dinv = dinv2.reshape(NP)
    y, x1 = _layer(y, rows2, cols2, dinv)
    y, x2 = _layer(y, rows2, cols2, dinv)
    _, x3 = _layer(y, rows2, cols2, dinv)

    final = _mean4(x0, x1, x2, x3)
    ur, ir = _gather(final, uids2, iids2)
    return _dot(ur, ir)
